# Initial kernel scaffold; baseline (speedup 1.0000x reference)
import jax
import jax.numpy as jnp
from jax.experimental import pallas as pl


def kernel(x, edge_index, edge_mask, W, a_src, a_dst):
    raise NotImplementedError("write your pallas kernel here")


# speedup vs baseline: 15.2917x; 15.2917x over previous
import functools

import jax
import jax.numpy as jnp
from jax import lax
from jax.experimental import pallas as pl
from jax.experimental.pallas import tpu as pltpu
from jax.experimental.pallas import tpu_sc as plsc

N = 10000
E = 320000
D = 128

DEN_W = 16
NW = 32
EPW = E // NW
CH = 80
NCHUNK = EPW // CH
ZR = 80
NZCHUNK = N // ZR
BLK = 1000


def _encode_body(x_ref, w_ref, a_ref, h_ref, sd_ref, c_ref, mx_ref):
    i = pl.program_id(0)
    h = jnp.dot(x_ref[...], w_ref[...], preferred_element_type=jnp.float32)
    h_ref[...] = h
    sd = jnp.dot(h, a_ref[...], preferred_element_type=jnp.float32)
    sd_ref[...] = sd
    ms = jnp.max(sd[:, 0])
    md = jnp.max(sd[:, 1])

    @pl.when(i == 0)
    def _():
        mx_ref[0] = ms
        mx_ref[1] = md

    @pl.when(i > 0)
    def _():
        mx_ref[0] = jnp.maximum(mx_ref[0], ms)
        mx_ref[1] = jnp.maximum(mx_ref[1], md)

    @pl.when(i == pl.num_programs(0) - 1)
    def _():
        c_ref[...] = jnp.full((1, 16), jnp.maximum(mx_ref[0] + mx_ref[1], 0.0),
                              jnp.float32)


def _encode(x, W, a2):
    return pl.pallas_call(
        _encode_body,
        grid=(N // BLK,),
        in_specs=[
            pl.BlockSpec((BLK, D), lambda i: (i, 0)),
            pl.BlockSpec((D, D), lambda i: (0, 0)),
            pl.BlockSpec((D, 2), lambda i: (0, 0)),
        ],
        out_specs=[
            pl.BlockSpec((BLK, D), lambda i: (i, 0)),
            pl.BlockSpec((BLK, 2), lambda i: (i, 0)),
            pl.BlockSpec((1, 16), lambda i: (0, 0)),
        ],
        out_shape=[
            jax.ShapeDtypeStruct((N, D), jnp.float32),
            jax.ShapeDtypeStruct((N, 2), jnp.float32),
            jax.ShapeDtypeStruct((1, 16), jnp.float32),
        ],
        scratch_shapes=[pltpu.SMEM((2,), jnp.float32)],
    )(x, W, a2)


def _edge_body(h_hbm, sd_hbm, src_hbm, dst_hbm, mask_hbm, c_hbm, zf_hbm,
               zd_hbm, accf_hbm, accd_hbm, sd_v, c_v, src_v, dst_v, mask_v,
               w_v, rows_v, den_v, accf_sh, accd_sh, sem):
    cid = lax.axis_index("c")
    sid = lax.axis_index("s")
    wid = sid * 2 + cid

    pltpu.sync_copy(sd_hbm, sd_v)
    pltpu.sync_copy(c_hbm, c_v)

    def zchunk(b, carry):
        ci = b * 16 + sid

        @pl.when(ci < NZCHUNK)
        def _():
            pltpu.sync_copy(zf_hbm, accf_sh.at[pl.ds(ci * ZR, ZR)])
            pltpu.sync_copy(zd_hbm, accd_sh.at[pl.ds(ci * ZR, ZR)])
        return carry

    lax.fori_loop(0, (NZCHUNK + 15) // 16, zchunk, 0)
    plsc.subcore_barrier()

    cvec = c_v[...]
    ebase = wid * EPW

    def chunk(ci, carry):
        off = ebase + ci * CH
        pltpu.sync_copy(src_hbm.at[pl.ds(off, CH)], src_v)
        pltpu.sync_copy(dst_hbm.at[pl.ds(off, CH)], dst_v)
        pltpu.sync_copy(mask_hbm.at[pl.ds(off, CH)], mask_v)
        gather = pltpu.async_copy(h_hbm.at[src_v], rows_v, sem)
        for g in range(CH // 16):
            s16 = src_v[pl.ds(g * 16, 16)]
            d16 = dst_v[pl.ds(g * 16, 16)]
            mk = mask_v[pl.ds(g * 16, 16)]
            sv = plsc.load_gather(sd_v, [s16 * 2])
            dv = plsc.load_gather(sd_v, [d16 * 2 + 1])
            e = sv + dv
            e = jnp.where(e >= 0.0, e, 0.2 * e)
            w16 = jnp.exp(e - cvec) / (1.0 + jnp.exp(-mk))
            w_v[pl.ds(g * 16, 16)] = w16
        gather.wait()
        lane0 = lax.iota(jnp.int32, 16) == 0
        for g in range(CH // 16):
            w16 = w_v[pl.ds(g * 16, 16)]
            for j in range(16):
                i = g * 16 + j
                wb = jnp.broadcast_to(w16[j], (16,))
                for k in range(D // 16):
                    rows_v[i, pl.ds(k * 16, 16)] = (
                        rows_v[i, pl.ds(k * 16, 16)] * wb)
                den_v[i, pl.ds(0, 16)] = jnp.where(lane0, wb, 0.0)
        pltpu.sync_copy(rows_v, accf_sh.at[dst_v], add=True)
        pltpu.sync_copy(den_v, accd_sh.at[dst_v], add=True)
        return carry

    lax.fori_loop(0, NCHUNK, chunk, 0)
    plsc.subcore_barrier()

    def fchunk(b, carry):
        ci = b * 16 + sid

        @pl.when(ci < NZCHUNK)
        def _():
            pltpu.sync_copy(accf_sh.at[pl.ds(ci * ZR, ZR)],
                            accf_hbm.at[cid, pl.ds(ci * ZR, ZR)])
            pltpu.sync_copy(accd_sh.at[pl.ds(ci * ZR, ZR)],
                            accd_hbm.at[cid, pl.ds(ci * ZR, ZR)])
        return carry

    lax.fori_loop(0, (NZCHUNK + 15) // 16, fchunk, 0)


def _edge_pass(h, sd_flat, src, dst, mask, cvec, zf, zd):
    mesh = plsc.VectorSubcoreMesh(core_axis_name="c", subcore_axis_name="s")
    f = functools.partial(
        pl.kernel,
        mesh=mesh,
        out_type=(
            jax.ShapeDtypeStruct((2, N, D), jnp.float32),
            jax.ShapeDtypeStruct((2, N, DEN_W), jnp.float32),
        ),
        scratch_types=[
            pltpu.VMEM((2 * N,), jnp.float32),
            pltpu.VMEM((16,), jnp.float32),
            pltpu.VMEM((CH,), jnp.int32),
            pltpu.VMEM((CH,), jnp.int32),
            pltpu.VMEM((CH,), jnp.float32),
            pltpu.VMEM((CH,), jnp.float32),
            pltpu.VMEM((CH, D), jnp.float32),
            pltpu.VMEM((CH, DEN_W), jnp.float32),
            pltpu.VMEM_SHARED((N, D), jnp.float32),
            pltpu.VMEM_SHARED((N, DEN_W), jnp.float32),
            pltpu.SemaphoreType.DMA,
        ],
        compiler_params=pltpu.CompilerParams(use_tc_tiling_on_sc=False,
                                             needs_layout_passes=False),
    )(_edge_body)
    return f(h, sd_flat, src, dst, mask, cvec, zf, zd)


def _finish_body(accf_ref, accd_ref, out_ref):
    num = accf_ref[0] + accf_ref[1]
    den = accd_ref[0, :, :1] + accd_ref[1, :, :1]
    out_ref[...] = num / (den + 1e-15)


def _finish(accf, accd):
    return pl.pallas_call(
        _finish_body,
        grid=(N // BLK,),
        in_specs=[
            pl.BlockSpec((2, BLK, D), lambda i: (0, i, 0)),
            pl.BlockSpec((2, BLK, DEN_W), lambda i: (0, i, 0)),
        ],
        out_specs=pl.BlockSpec((BLK, D), lambda i: (i, 0)),
        out_shape=jax.ShapeDtypeStruct((N, D), jnp.float32),
    )(accf, accd)


def kernel(x, edge_index, edge_mask, W, a_src, a_dst):
    a2 = jnp.stack([a_src, a_dst], axis=1)
    h, sd, cvec = _encode(x, W, a2)
    src = edge_index[0]
    dst = edge_index[1]
    zf = jnp.zeros((ZR, D), jnp.float32)
    zd = jnp.zeros((ZR, DEN_W), jnp.float32)
    accf, accd = _edge_pass(h, sd.reshape(2 * N), src, dst, edge_mask,
                            cvec.reshape(16), zf, zd)
    return _finish(accf, accd)


# speedup vs baseline: 21.8319x; 1.4277x over previous
import functools

import jax
import jax.numpy as jnp
from jax import lax
from jax.experimental import pallas as pl
from jax.experimental.pallas import tpu as pltpu
from jax.experimental.pallas import tpu_sc as plsc

N = 10000
E = 320000
D = 128

DEN_W = 16
NW = 32
EPW = E // NW
CH = 80
NCHUNK = EPW // CH
ZR = 80
NZCHUNK = N // ZR
BLK = 1000


def _encode_body(x_ref, w_ref, a_ref, h_ref, sd_ref, c_ref, mx_ref):
    i = pl.program_id(0)
    h = jnp.dot(x_ref[...], w_ref[...], preferred_element_type=jnp.float32)
    h_ref[...] = h
    sd = jnp.dot(h, a_ref[...], preferred_element_type=jnp.float32)
    sd_ref[...] = sd
    ms = jnp.max(sd[:, 0])
    md = jnp.max(sd[:, 1])

    @pl.when(i == 0)
    def _():
        mx_ref[0] = ms
        mx_ref[1] = md

    @pl.when(i > 0)
    def _():
        mx_ref[0] = jnp.maximum(mx_ref[0], ms)
        mx_ref[1] = jnp.maximum(mx_ref[1], md)

    @pl.when(i == pl.num_programs(0) - 1)
    def _():
        c_ref[...] = jnp.full((1, 16), jnp.maximum(mx_ref[0] + mx_ref[1], 0.0),
                              jnp.float32)


def _encode(x, W, a2):
    return pl.pallas_call(
        _encode_body,
        grid=(N // BLK,),
        in_specs=[
            pl.BlockSpec((BLK, D), lambda i: (i, 0)),
            pl.BlockSpec((D, D), lambda i: (0, 0)),
            pl.BlockSpec((D, 2), lambda i: (0, 0)),
        ],
        out_specs=[
            pl.BlockSpec((BLK, D), lambda i: (i, 0)),
            pl.BlockSpec((BLK, 2), lambda i: (i, 0)),
            pl.BlockSpec((1, 16), lambda i: (0, 0)),
        ],
        out_shape=[
            jax.ShapeDtypeStruct((N, D), jnp.float32),
            jax.ShapeDtypeStruct((N, 2), jnp.float32),
            jax.ShapeDtypeStruct((1, 16), jnp.float32),
        ],
        scratch_shapes=[pltpu.SMEM((2,), jnp.float32)],
    )(x, W, a2)


SBC = 25
NBLK = NCHUNK // SBC


def _edge_body(h_hbm, sd_hbm, src_hbm, dst_hbm, mask_hbm, c_hbm, zf_hbm,
               zd_hbm, accf_hbm, accd_hbm, sd_v, c_v, srcb_v, dstb_v,
               maskb_v, w_v, rows_v, den_v, accf_sh, accd_sh, gsem, ssem):
    cid = lax.axis_index("c")
    sid = lax.axis_index("s")
    wid = sid * 2 + cid

    pltpu.sync_copy(sd_hbm, sd_v)
    pltpu.sync_copy(c_hbm, c_v)

    def zchunk(b, carry):
        ci = b * 16 + sid

        @pl.when(ci < NZCHUNK)
        def _():
            pltpu.sync_copy(zf_hbm, accf_sh.at[pl.ds(ci * ZR, ZR)])
            pltpu.sync_copy(zd_hbm, accd_sh.at[pl.ds(ci * ZR, ZR)])
        return carry

    lax.fori_loop(0, (NZCHUNK + 15) // 16, zchunk, 0)
    plsc.subcore_barrier()

    cvec = c_v[...]
    crow0 = wid * NCHUNK
    lane0 = lax.iota(jnp.int32, 16) == 0

    def drain_scatter():
        pltpu.make_async_copy(rows_v, accf_sh.at[dstb_v.at[0]], ssem).wait()
        pltpu.make_async_copy(den_v, accd_sh.at[dstb_v.at[0]], ssem).wait()

    def process(lci):
        pltpu.async_copy(h_hbm.at[srcb_v.at[lci]], rows_v, gsem)
        for g in range(CH // 16):
            s16 = srcb_v[lci, pl.ds(g * 16, 16)]
            d16 = dstb_v[lci, pl.ds(g * 16, 16)]
            mk = maskb_v[lci, pl.ds(g * 16, 16)]
            sv = plsc.load_gather(sd_v, [s16 * 2])
            dv = plsc.load_gather(sd_v, [d16 * 2 + 1])
            e = sv + dv
            e = jnp.where(e >= 0.0, e, 0.2 * e)
            w16 = jnp.exp(e - cvec) / (1.0 + jnp.exp(-mk))
            w_v[pl.ds(g * 16, 16)] = w16
        pltpu.make_async_copy(h_hbm.at[srcb_v.at[lci]], rows_v, gsem).wait()
        for g in range(CH // 16):
            w16 = w_v[pl.ds(g * 16, 16)]
            for j in range(16):
                i = g * 16 + j
                wb = jnp.broadcast_to(w16[j], (16,))
                for k in range(D // 16):
                    rows_v[i, pl.ds(k * 16, 16)] = (
                        rows_v[i, pl.ds(k * 16, 16)] * wb)
                den_v[i, pl.ds(0, 16)] = jnp.where(lane0, wb, 0.0)
        pltpu.async_copy(rows_v, accf_sh.at[dstb_v.at[lci]], ssem, add=True)
        pltpu.async_copy(den_v, accd_sh.at[dstb_v.at[lci]], ssem, add=True)

    def block(blk, carry):
        @pl.when(blk > 0)
        def _():
            drain_scatter()

        brow = crow0 + blk * SBC
        pltpu.sync_copy(src_hbm.at[pl.ds(brow, SBC)], srcb_v)
        pltpu.sync_copy(dst_hbm.at[pl.ds(brow, SBC)], dstb_v)
        pltpu.sync_copy(mask_hbm.at[pl.ds(brow, SBC)], maskb_v)

        def chunk(lci, c2):
            @pl.when(lci > 0)
            def _():
                drain_scatter()
            process(lci)
            return c2

        lax.fori_loop(0, SBC, chunk, 0)
        return carry

    lax.fori_loop(0, NBLK, block, 0)
    drain_scatter()
    plsc.subcore_barrier()

    def fchunk(b, carry):
        ci = b * 16 + sid

        @pl.when(ci < NZCHUNK)
        def _():
            pltpu.sync_copy(accf_sh.at[pl.ds(ci * ZR, ZR)],
                            accf_hbm.at[cid, pl.ds(ci * ZR, ZR)])
            pltpu.sync_copy(accd_sh.at[pl.ds(ci * ZR, ZR)],
                            accd_hbm.at[cid, pl.ds(ci * ZR, ZR)])
        return carry

    lax.fori_loop(0, (NZCHUNK + 15) // 16, fchunk, 0)


def _edge_pass(h, sd_flat, src, dst, mask, cvec, zf, zd):
    mesh = plsc.VectorSubcoreMesh(core_axis_name="c", subcore_axis_name="s")
    f = functools.partial(
        pl.kernel,
        mesh=mesh,
        out_type=(
            jax.ShapeDtypeStruct((2, N, D), jnp.float32),
            jax.ShapeDtypeStruct((2, N, DEN_W), jnp.float32),
        ),
        scratch_types=[
            pltpu.VMEM((2 * N,), jnp.float32),
            pltpu.VMEM((16,), jnp.float32),
            pltpu.VMEM((SBC, CH), jnp.int32),
            pltpu.VMEM((SBC, CH), jnp.int32),
            pltpu.VMEM((SBC, CH), jnp.float32),
            pltpu.VMEM((CH,), jnp.float32),
            pltpu.VMEM((CH, D), jnp.float32),
            pltpu.VMEM((CH, DEN_W), jnp.float32),
            pltpu.VMEM_SHARED((N, D), jnp.float32),
            pltpu.VMEM_SHARED((N, DEN_W), jnp.float32),
            pltpu.SemaphoreType.DMA,
            pltpu.SemaphoreType.DMA,
        ],
        compiler_params=pltpu.CompilerParams(use_tc_tiling_on_sc=False,
                                             needs_layout_passes=False),
    )(_edge_body)
    return f(h, sd_flat, src, dst, mask, cvec, zf, zd)


def _finish_body(accf_ref, accd_ref, out_ref):
    num = accf_ref[0] + accf_ref[1]
    den = accd_ref[0, :, :1] + accd_ref[1, :, :1]
    out_ref[...] = num / (den + 1e-15)


def _finish(accf, accd):
    return pl.pallas_call(
        _finish_body,
        grid=(N // BLK,),
        in_specs=[
            pl.BlockSpec((2, BLK, D), lambda i: (0, i, 0)),
            pl.BlockSpec((2, BLK, DEN_W), lambda i: (0, i, 0)),
        ],
        out_specs=pl.BlockSpec((BLK, D), lambda i: (i, 0)),
        out_shape=jax.ShapeDtypeStruct((N, D), jnp.float32),
    )(accf, accd)


def kernel(x, edge_index, edge_mask, W, a_src, a_dst):
    a2 = jnp.stack([a_src, a_dst], axis=1)
    h, sd, cvec = _encode(x, W, a2)
    src = edge_index[0].reshape(E // CH, CH)
    dst = edge_index[1].reshape(E // CH, CH)
    mask2 = edge_mask.reshape(E // CH, CH)
    zf = jnp.zeros((ZR, D), jnp.float32)
    zd = jnp.zeros((ZR, DEN_W), jnp.float32)
    accf, accd = _edge_pass(h, sd.reshape(2 * N), src, dst, mask2,
                            cvec.reshape(16), zf, zd)
    return _finish(accf, accd)


# speedup vs baseline: 22.1452x; 1.0143x over previous
import functools

import jax
import jax.numpy as jnp
from jax import lax
from jax.experimental import pallas as pl
from jax.experimental.pallas import tpu as pltpu
from jax.experimental.pallas import tpu_sc as plsc

N = 10000
E = 320000
D = 128

DEN_W = 16
NW = 32
EPW = E // NW
CH = 80
NCHUNK = EPW // CH
ZR = 80
NZCHUNK = N // ZR
BLK = 1000


def _encode_body(x_ref, w_ref, a_ref, h_ref, sd_ref, c_ref, mx_ref):
    i = pl.program_id(0)
    h = jnp.dot(x_ref[...], w_ref[...], preferred_element_type=jnp.float32)
    h_ref[...] = h
    sd = jnp.dot(h, a_ref[...], preferred_element_type=jnp.float32)
    sd_ref[...] = sd
    ms = jnp.max(sd[:, 0])
    md = jnp.max(sd[:, 1])

    @pl.when(i == 0)
    def _():
        mx_ref[0] = ms
        mx_ref[1] = md

    @pl.when(i > 0)
    def _():
        mx_ref[0] = jnp.maximum(mx_ref[0], ms)
        mx_ref[1] = jnp.maximum(mx_ref[1], md)

    @pl.when(i == pl.num_programs(0) - 1)
    def _():
        c_ref[...] = jnp.full((1, 16), jnp.maximum(mx_ref[0] + mx_ref[1], 0.0),
                              jnp.float32)


def _encode(x, W, a2):
    return pl.pallas_call(
        _encode_body,
        grid=(N // BLK,),
        in_specs=[
            pl.BlockSpec((BLK, D), lambda i: (i, 0)),
            pl.BlockSpec((D, D), lambda i: (0, 0)),
            pl.BlockSpec((D, 2), lambda i: (0, 0)),
        ],
        out_specs=[
            pl.BlockSpec((BLK, D), lambda i: (i, 0)),
            pl.BlockSpec((BLK, 2), lambda i: (i, 0)),
            pl.BlockSpec((1, 16), lambda i: (0, 0)),
        ],
        out_shape=[
            jax.ShapeDtypeStruct((N, D), jnp.float32),
            jax.ShapeDtypeStruct((N, 2), jnp.float32),
            jax.ShapeDtypeStruct((1, 16), jnp.float32),
        ],
        scratch_shapes=[pltpu.SMEM((2,), jnp.float32)],
    )(x, W, a2)


SBC = 25
NBLK = NCHUNK // SBC


def _edge_body(h_hbm, sd_hbm, src_hbm, dst_hbm, mask_hbm, c_hbm, zf_hbm,
               zd_hbm, accf_hbm, accd_hbm, sd_v, c_v, srcb_v, dstb_v,
               maskb_v, w_v, rows_v, accf_sh, accd_sh, gsem, ssem):
    cid = lax.axis_index("c")
    sid = lax.axis_index("s")
    wid = sid * 2 + cid

    pltpu.sync_copy(sd_hbm, sd_v)
    pltpu.sync_copy(c_hbm, c_v)

    def zchunk(b, carry):
        ci = b * 16 + sid

        @pl.when(ci < NZCHUNK)
        def _():
            pltpu.sync_copy(zf_hbm, accf_sh.at[pl.ds(ci * ZR, ZR)])
            pltpu.sync_copy(zd_hbm, accd_sh.at[pl.ds(ci * ZR, ZR)])
        return carry

    lax.fori_loop(0, (NZCHUNK + 15) // 16, zchunk, 0)
    plsc.subcore_barrier()

    cvec = c_v[...]
    crow0 = wid * NCHUNK

    def drain_scatter():
        pltpu.make_async_copy(rows_v, accf_sh.at[dstb_v.at[0]], ssem).wait()
        pltpu.make_async_copy(w_v, accd_sh.at[dstb_v.at[0]], ssem).wait()

    def process(lci):
        pltpu.async_copy(h_hbm.at[srcb_v.at[lci]], rows_v, gsem)
        for g in range(CH // 16):
            s16 = srcb_v[lci, pl.ds(g * 16, 16)]
            d16 = dstb_v[lci, pl.ds(g * 16, 16)]
            mk = maskb_v[lci, pl.ds(g * 16, 16)]
            sv = plsc.load_gather(sd_v, [s16 * 2])
            dv = plsc.load_gather(sd_v, [d16 * 2 + 1])
            e = sv + dv
            e = jnp.where(e >= 0.0, e, 0.2 * e)
            w16 = jnp.exp(e - cvec) / (1.0 + jnp.exp(-mk))
            w_v[pl.ds(g * 16, 16)] = w16
        pltpu.make_async_copy(h_hbm.at[srcb_v.at[lci]], rows_v, gsem).wait()
        for g in range(CH // 16):
            w16 = w_v[pl.ds(g * 16, 16)]
            for j in range(16):
                i = g * 16 + j
                wb = w16[j]
                for k in range(D // 16):
                    rows_v[i, pl.ds(k * 16, 16)] = (
                        rows_v[i, pl.ds(k * 16, 16)] * wb)
        pltpu.async_copy(rows_v, accf_sh.at[dstb_v.at[lci]], ssem, add=True)
        pltpu.async_copy(w_v, accd_sh.at[dstb_v.at[lci]], ssem, add=True)

    def block(blk, carry):
        @pl.when(blk > 0)
        def _():
            drain_scatter()

        brow = crow0 + blk * SBC
        pltpu.sync_copy(src_hbm.at[pl.ds(brow, SBC)], srcb_v)
        pltpu.sync_copy(dst_hbm.at[pl.ds(brow, SBC)], dstb_v)
        pltpu.sync_copy(mask_hbm.at[pl.ds(brow, SBC)], maskb_v)

        def chunk(lci, c2):
            @pl.when(lci > 0)
            def _():
                drain_scatter()
            process(lci)
            return c2

        lax.fori_loop(0, SBC, chunk, 0)
        return carry

    lax.fori_loop(0, NBLK, block, 0)
    drain_scatter()
    plsc.subcore_barrier()

    def fchunk(b, carry):
        ci = b * 16 + sid

        @pl.when(ci < NZCHUNK)
        def _():
            pltpu.sync_copy(accf_sh.at[pl.ds(ci * ZR, ZR)],
                            accf_hbm.at[cid, pl.ds(ci * ZR, ZR)])
            pltpu.sync_copy(accd_sh.at[pl.ds(ci * ZR, ZR)],
                            accd_hbm.at[cid, pl.ds(ci * ZR, ZR)])
        return carry

    lax.fori_loop(0, (NZCHUNK + 15) // 16, fchunk, 0)


def _edge_pass(h, sd_flat, src, dst, mask, cvec, zf, zd):
    mesh = plsc.VectorSubcoreMesh(core_axis_name="c", subcore_axis_name="s")
    f = functools.partial(
        pl.kernel,
        mesh=mesh,
        out_type=(
            jax.ShapeDtypeStruct((2, N, D), jnp.float32),
            jax.ShapeDtypeStruct((2, N), jnp.float32),
        ),
        scratch_types=[
            pltpu.VMEM((2 * N,), jnp.float32),
            pltpu.VMEM((16,), jnp.float32),
            pltpu.VMEM((SBC, CH), jnp.int32),
            pltpu.VMEM((SBC, CH), jnp.int32),
            pltpu.VMEM((SBC, CH), jnp.float32),
            pltpu.VMEM((CH,), jnp.float32),
            pltpu.VMEM((CH, D), jnp.float32),
            pltpu.VMEM_SHARED((N, D), jnp.float32),
            pltpu.VMEM_SHARED((N,), jnp.float32),
            pltpu.SemaphoreType.DMA,
            pltpu.SemaphoreType.DMA,
        ],
        compiler_params=pltpu.CompilerParams(use_tc_tiling_on_sc=False,
                                             needs_layout_passes=False),
    )(_edge_body)
    return f(h, sd_flat, src, dst, mask, cvec, zf, zd)


def _finish_body(accf_ref, accd_ref, out_ref):
    num = accf_ref[0] + accf_ref[1]
    den = accd_ref[0] + accd_ref[1]
    out_ref[...] = num / (den + 1e-15)


def _finish(accf, accd):
    return pl.pallas_call(
        _finish_body,
        grid=(N // BLK,),
        in_specs=[
            pl.BlockSpec((2, BLK, D), lambda i: (0, i, 0)),
            pl.BlockSpec((2, BLK, 1), lambda i: (0, i, 0)),
        ],
        out_specs=pl.BlockSpec((BLK, D), lambda i: (i, 0)),
        out_shape=jax.ShapeDtypeStruct((N, D), jnp.float32),
    )(accf, accd)


def kernel(x, edge_index, edge_mask, W, a_src, a_dst):
    a2 = jnp.stack([a_src, a_dst], axis=1)
    h, sd, cvec = _encode(x, W, a2)
    src = edge_index[0].reshape(E // CH, CH)
    dst = edge_index[1].reshape(E // CH, CH)
    mask2 = edge_mask.reshape(E // CH, CH)
    zf = jnp.zeros((ZR, D), jnp.float32)
    zd = jnp.zeros((ZR,), jnp.float32)
    accf, accd = _edge_pass(h, sd.reshape(2 * N), src, dst, mask2,
                            cvec.reshape(16), zf, zd)
    return _finish(accf, accd.reshape(2, N, 1))


# speedup vs baseline: 23.1197x; 1.0440x over previous
import functools

import jax
import jax.numpy as jnp
from jax import lax
from jax.experimental import pallas as pl
from jax.experimental.pallas import tpu as pltpu
from jax.experimental.pallas import tpu_sc as plsc

N = 10000
E = 320000
D = 128

DEN_W = 16
NW = 32
EPW = E // NW
CH = 80
NCHUNK = EPW // CH
ZR = 80
NZCHUNK = N // ZR
BLK = 1000


def _encode_body(x_ref, w_ref, a_ref, h_ref, sd_ref, c_ref, mx_ref):
    i = pl.program_id(0)
    h = jnp.dot(x_ref[...], w_ref[...], preferred_element_type=jnp.float32)
    h_ref[...] = h
    sd = jnp.dot(h, a_ref[...], preferred_element_type=jnp.float32)
    sd_ref[...] = sd
    ms = jnp.max(sd[:, 0])
    md = jnp.max(sd[:, 1])

    @pl.when(i == 0)
    def _():
        mx_ref[0] = ms
        mx_ref[1] = md

    @pl.when(i > 0)
    def _():
        mx_ref[0] = jnp.maximum(mx_ref[0], ms)
        mx_ref[1] = jnp.maximum(mx_ref[1], md)

    @pl.when(i == pl.num_programs(0) - 1)
    def _():
        c_ref[...] = jnp.full((1, 16), jnp.maximum(mx_ref[0] + mx_ref[1], 0.0),
                              jnp.float32)


def _encode(x, W, a2):
    return pl.pallas_call(
        _encode_body,
        grid=(N // BLK,),
        in_specs=[
            pl.BlockSpec((BLK, D), lambda i: (i, 0)),
            pl.BlockSpec((D, D), lambda i: (0, 0)),
            pl.BlockSpec((D, 2), lambda i: (0, 0)),
        ],
        out_specs=[
            pl.BlockSpec((BLK, D), lambda i: (i, 0)),
            pl.BlockSpec((BLK, 2), lambda i: (i, 0)),
            pl.BlockSpec((1, 16), lambda i: (0, 0)),
        ],
        out_shape=[
            jax.ShapeDtypeStruct((N, D), jnp.float32),
            jax.ShapeDtypeStruct((N, 2), jnp.float32),
            jax.ShapeDtypeStruct((1, 16), jnp.float32),
        ],
        scratch_shapes=[pltpu.SMEM((2,), jnp.float32)],
    )(x, W, a2)


SBC = 25
NBLK = NCHUNK // SBC


def _edge_body(h_hbm, sd_hbm, src_hbm, dst_hbm, mask_hbm, c_hbm, zf_hbm,
               zd_hbm, accf_hbm, accd_hbm, sd_v, c_v, srcb_v, dstb_v,
               maskb_v, w0_v, w1_v, rows0_v, rows1_v, accf_sh, accd_sh,
               gsem0, gsem1, ssem0, ssem1):
    cid = lax.axis_index("c")
    sid = lax.axis_index("s")
    wid = sid * 2 + cid

    pltpu.sync_copy(sd_hbm, sd_v)
    pltpu.sync_copy(c_hbm, c_v)

    def zchunk(b, carry):
        ci = b * 16 + sid

        @pl.when(ci < NZCHUNK)
        def _():
            pltpu.sync_copy(zf_hbm, accf_sh.at[pl.ds(ci * ZR, ZR)])
            pltpu.sync_copy(zd_hbm, accd_sh.at[pl.ds(ci * ZR, ZR)])
        return carry

    lax.fori_loop(0, (NZCHUNK + 15) // 16, zchunk, 0)
    plsc.subcore_barrier()

    cvec = c_v[...]
    crow0 = wid * NCHUNK

    def fire_gather(lci, rows_v, gsem):
        pltpu.async_copy(h_hbm.at[srcb_v.at[lci]], rows_v, gsem)

    def drain_scatter(rows_v, w_v, ssem):
        pltpu.make_async_copy(rows_v, accf_sh.at[dstb_v.at[0]], ssem).wait()
        pltpu.make_async_copy(w_v, accd_sh.at[dstb_v.at[0]], ssem).wait()

    def process(lci, rows_v, w_v, gsem, ssem):
        for g in range(CH // 16):
            s16 = srcb_v[lci, pl.ds(g * 16, 16)]
            d16 = dstb_v[lci, pl.ds(g * 16, 16)]
            mk = maskb_v[lci, pl.ds(g * 16, 16)]
            sv = plsc.load_gather(sd_v, [s16 * 2])
            dv = plsc.load_gather(sd_v, [d16 * 2 + 1])
            e = sv + dv
            e = jnp.where(e >= 0.0, e, 0.2 * e)
            w16 = jnp.exp(e - cvec) / (1.0 + jnp.exp(-mk))
            w_v[pl.ds(g * 16, 16)] = w16
        pltpu.make_async_copy(h_hbm.at[srcb_v.at[lci]], rows_v, gsem).wait()
        for g in range(CH // 16):
            w16 = w_v[pl.ds(g * 16, 16)]
            for j in range(16):
                i = g * 16 + j
                wb = w16[j]
                for k in range(D // 16):
                    rows_v[i, pl.ds(k * 16, 16)] = (
                        rows_v[i, pl.ds(k * 16, 16)] * wb)
        pltpu.async_copy(rows_v, accf_sh.at[dstb_v.at[lci]], ssem, add=True)
        pltpu.async_copy(w_v, accd_sh.at[dstb_v.at[lci]], ssem, add=True)

    def block(blk, carry):
        @pl.when(blk > 0)
        def _():
            drain_scatter(rows0_v, w0_v, ssem0)
            drain_scatter(rows1_v, w1_v, ssem1)

        brow = crow0 + blk * SBC
        pltpu.sync_copy(src_hbm.at[pl.ds(brow, SBC)], srcb_v)
        pltpu.sync_copy(dst_hbm.at[pl.ds(brow, SBC)], dstb_v)
        pltpu.sync_copy(mask_hbm.at[pl.ds(brow, SBC)], maskb_v)

        def pair(p, c2):
            lci0 = p * 2
            lci1 = p * 2 + 1
            notfirst = p > 0

            @pl.when(notfirst)
            def _():
                drain_scatter(rows0_v, w0_v, ssem0)
            fire_gather(lci0, rows0_v, gsem0)

            @pl.when(notfirst)
            def _():
                drain_scatter(rows1_v, w1_v, ssem1)
            fire_gather(lci1, rows1_v, gsem1)

            process(lci0, rows0_v, w0_v, gsem0, ssem0)
            process(lci1, rows1_v, w1_v, gsem1, ssem1)
            return c2

        lax.fori_loop(0, SBC // 2, pair, 0)

        drain_scatter(rows0_v, w0_v, ssem0)
        fire_gather(SBC - 1, rows0_v, gsem0)
        process(SBC - 1, rows0_v, w0_v, gsem0, ssem0)
        return carry

    lax.fori_loop(0, NBLK, block, 0)
    drain_scatter(rows0_v, w0_v, ssem0)
    drain_scatter(rows1_v, w1_v, ssem1)
    plsc.subcore_barrier()

    def fchunk(b, carry):
        ci = b * 16 + sid

        @pl.when(ci < NZCHUNK)
        def _():
            pltpu.sync_copy(accf_sh.at[pl.ds(ci * ZR, ZR)],
                            accf_hbm.at[cid, pl.ds(ci * ZR, ZR)])
            pltpu.sync_copy(accd_sh.at[pl.ds(ci * ZR, ZR)],
                            accd_hbm.at[cid, pl.ds(ci * ZR, ZR)])
        return carry

    lax.fori_loop(0, (NZCHUNK + 15) // 16, fchunk, 0)


def _edge_pass(h, sd_flat, src, dst, mask, cvec, zf, zd):
    mesh = plsc.VectorSubcoreMesh(core_axis_name="c", subcore_axis_name="s")
    f = functools.partial(
        pl.kernel,
        mesh=mesh,
        out_type=(
            jax.ShapeDtypeStruct((2, N, D), jnp.float32),
            jax.ShapeDtypeStruct((2, N), jnp.float32),
        ),
        scratch_types=[
            pltpu.VMEM((2 * N,), jnp.float32),
            pltpu.VMEM((16,), jnp.float32),
            pltpu.VMEM((SBC, CH), jnp.int32),
            pltpu.VMEM((SBC, CH), jnp.int32),
            pltpu.VMEM((SBC, CH), jnp.float32),
            pltpu.VMEM((CH,), jnp.float32),
            pltpu.VMEM((CH,), jnp.float32),
            pltpu.VMEM((CH, D), jnp.float32),
            pltpu.VMEM((CH, D), jnp.float32),
            pltpu.VMEM_SHARED((N, D), jnp.float32),
            pltpu.VMEM_SHARED((N,), jnp.float32),
            pltpu.SemaphoreType.DMA,
            pltpu.SemaphoreType.DMA,
            pltpu.SemaphoreType.DMA,
            pltpu.SemaphoreType.DMA,
        ],
        compiler_params=pltpu.CompilerParams(use_tc_tiling_on_sc=False,
                                             needs_layout_passes=False),
    )(_edge_body)
    return f(h, sd_flat, src, dst, mask, cvec, zf, zd)


def _finish_body(accf_ref, accd_ref, out_ref):
    num = accf_ref[0] + accf_ref[1]
    den = accd_ref[0] + accd_ref[1]
    out_ref[...] = num / (den + 1e-15)


def _finish(accf, accd):
    return pl.pallas_call(
        _finish_body,
        grid=(N // BLK,),
        in_specs=[
            pl.BlockSpec((2, BLK, D), lambda i: (0, i, 0)),
            pl.BlockSpec((2, BLK, 1), lambda i: (0, i, 0)),
        ],
        out_specs=pl.BlockSpec((BLK, D), lambda i: (i, 0)),
        out_shape=jax.ShapeDtypeStruct((N, D), jnp.float32),
    )(accf, accd)


def kernel(x, edge_index, edge_mask, W, a_src, a_dst):
    a2 = jnp.stack([a_src, a_dst], axis=1)
    h, sd, cvec = _encode(x, W, a2)
    src = edge_index[0].reshape(E // CH, CH)
    dst = edge_index[1].reshape(E // CH, CH)
    mask2 = edge_mask.reshape(E // CH, CH)
    zf = jnp.zeros((ZR, D), jnp.float32)
    zd = jnp.zeros((ZR,), jnp.float32)
    accf, accd = _edge_pass(h, sd.reshape(2 * N), src, dst, mask2,
                            cvec.reshape(16), zf, zd)
    return _finish(accf, accd.reshape(2, N, 1))


# speedup vs baseline: 23.3050x; 1.0080x over previous
import functools

import jax
import jax.numpy as jnp
from jax import lax
from jax.experimental import pallas as pl
from jax.experimental.pallas import tpu as pltpu
from jax.experimental.pallas import tpu_sc as plsc

N = 10000
E = 320000
D = 128

DEN_W = 16
NW = 32
EPW = E // NW
CH = 80
NCHUNK = EPW // CH
ZR = 80
NZCHUNK = N // ZR
BLK = 1000


def _encode_body(x_ref, w_ref, a_ref, h_ref, sd_ref, c_ref, mx_ref):
    i = pl.program_id(0)
    h = jnp.dot(x_ref[...], w_ref[...], preferred_element_type=jnp.float32)
    h_ref[...] = h
    sd = jnp.dot(h, a_ref[...], preferred_element_type=jnp.float32)
    sd_ref[...] = sd
    ms = jnp.max(sd[:, 0])
    md = jnp.max(sd[:, 1])

    @pl.when(i == 0)
    def _():
        mx_ref[0] = ms
        mx_ref[1] = md

    @pl.when(i > 0)
    def _():
        mx_ref[0] = jnp.maximum(mx_ref[0], ms)
        mx_ref[1] = jnp.maximum(mx_ref[1], md)

    @pl.when(i == pl.num_programs(0) - 1)
    def _():
        c_ref[...] = jnp.full((1, 16), jnp.maximum(mx_ref[0] + mx_ref[1], 0.0),
                              jnp.float32)


def _encode(x, W, a2):
    return pl.pallas_call(
        _encode_body,
        grid=(N // BLK,),
        in_specs=[
            pl.BlockSpec((BLK, D), lambda i: (i, 0)),
            pl.BlockSpec((D, D), lambda i: (0, 0)),
            pl.BlockSpec((D, 2), lambda i: (0, 0)),
        ],
        out_specs=[
            pl.BlockSpec((BLK, D), lambda i: (i, 0)),
            pl.BlockSpec((BLK, 2), lambda i: (i, 0)),
            pl.BlockSpec((1, 16), lambda i: (0, 0)),
        ],
        out_shape=[
            jax.ShapeDtypeStruct((N, D), jnp.float32),
            jax.ShapeDtypeStruct((N, 2), jnp.float32),
            jax.ShapeDtypeStruct((1, 16), jnp.float32),
        ],
        scratch_shapes=[pltpu.SMEM((2,), jnp.float32)],
    )(x, W, a2)


SBC = 25
NBLK = NCHUNK // SBC


def _edge_body(h_hbm, sd_hbm, src_hbm, dst_hbm, mask_hbm, c_hbm, zf_hbm,
               zd_hbm, accf_hbm, accd_hbm, sd_v, c_v, srcb_v, dstb_v,
               maskb_v, w0_v, w1_v, rows0_v, rows1_v, accf_sh, accd_sh,
               gsem0, gsem1, ssem0, ssem1):
    cid = lax.axis_index("c")
    sid = lax.axis_index("s")
    wid = sid * 2 + cid

    pltpu.async_copy(sd_hbm, sd_v, gsem0)
    pltpu.async_copy(c_hbm, c_v, gsem1)

    def zchunk(b, carry):
        ci = b * 16 + sid

        @pl.when(ci < NZCHUNK)
        def _():
            pltpu.async_copy(zf_hbm, accf_sh.at[pl.ds(ci * ZR, ZR)], ssem0)
            pltpu.async_copy(zd_hbm, accd_sh.at[pl.ds(ci * ZR, ZR)], ssem1)
        return carry

    def zwait(b, carry):
        ci = b * 16 + sid

        @pl.when(ci < NZCHUNK)
        def _():
            pltpu.make_async_copy(
                zf_hbm, accf_sh.at[pl.ds(ci * ZR, ZR)], ssem0).wait()
            pltpu.make_async_copy(
                zd_hbm, accd_sh.at[pl.ds(ci * ZR, ZR)], ssem1).wait()
        return carry

    lax.fori_loop(0, (NZCHUNK + 15) // 16, zchunk, 0)
    lax.fori_loop(0, (NZCHUNK + 15) // 16, zwait, 0)
    pltpu.make_async_copy(sd_hbm, sd_v, gsem0).wait()
    pltpu.make_async_copy(c_hbm, c_v, gsem1).wait()
    plsc.subcore_barrier()

    cvec = c_v[...]
    crow0 = wid * NCHUNK

    def fire_gather(lci, rows_v, gsem):
        pltpu.async_copy(h_hbm.at[srcb_v.at[lci]], rows_v, gsem)

    def drain_scatter(rows_v, w_v, ssem):
        pltpu.make_async_copy(rows_v, accf_sh.at[dstb_v.at[0]], ssem).wait()
        pltpu.make_async_copy(w_v, accd_sh.at[dstb_v.at[0]], ssem).wait()

    def process(lci, rows_v, w_v, gsem, ssem):
        for g in range(CH // 16):
            s16 = srcb_v[lci, pl.ds(g * 16, 16)]
            d16 = dstb_v[lci, pl.ds(g * 16, 16)]
            mk = maskb_v[lci, pl.ds(g * 16, 16)]
            sv = plsc.load_gather(sd_v, [s16 * 2])
            dv = plsc.load_gather(sd_v, [d16 * 2 + 1])
            e = sv + dv
            e = jnp.where(e >= 0.0, e, 0.2 * e)
            w16 = jnp.exp(e - cvec) / (1.0 + jnp.exp(-mk))
            w_v[pl.ds(g * 16, 16)] = w16
        pltpu.make_async_copy(h_hbm.at[srcb_v.at[lci]], rows_v, gsem).wait()
        for g in range(CH // 16):
            w16 = w_v[pl.ds(g * 16, 16)]
            for j in range(16):
                i = g * 16 + j
                wb = w16[j]
                for k in range(D // 16):
                    rows_v[i, pl.ds(k * 16, 16)] = (
                        rows_v[i, pl.ds(k * 16, 16)] * wb)
        pltpu.async_copy(rows_v, accf_sh.at[dstb_v.at[lci]], ssem, add=True)
        pltpu.async_copy(w_v, accd_sh.at[dstb_v.at[lci]], ssem, add=True)

    def block(blk, carry):
        @pl.when(blk > 0)
        def _():
            drain_scatter(rows0_v, w0_v, ssem0)
            drain_scatter(rows1_v, w1_v, ssem1)

        brow = crow0 + blk * SBC
        pltpu.sync_copy(src_hbm.at[pl.ds(brow, SBC)], srcb_v)
        pltpu.sync_copy(dst_hbm.at[pl.ds(brow, SBC)], dstb_v)
        pltpu.sync_copy(mask_hbm.at[pl.ds(brow, SBC)], maskb_v)

        def pair(p, c2):
            lci0 = p * 2
            lci1 = p * 2 + 1
            notfirst = p > 0

            @pl.when(notfirst)
            def _():
                drain_scatter(rows0_v, w0_v, ssem0)
            fire_gather(lci0, rows0_v, gsem0)

            @pl.when(notfirst)
            def _():
                drain_scatter(rows1_v, w1_v, ssem1)
            fire_gather(lci1, rows1_v, gsem1)

            process(lci0, rows0_v, w0_v, gsem0, ssem0)
            process(lci1, rows1_v, w1_v, gsem1, ssem1)
            return c2

        lax.fori_loop(0, SBC // 2, pair, 0)

        drain_scatter(rows0_v, w0_v, ssem0)
        fire_gather(SBC - 1, rows0_v, gsem0)
        process(SBC - 1, rows0_v, w0_v, gsem0, ssem0)
        return carry

    lax.fori_loop(0, NBLK, block, 0)
    drain_scatter(rows0_v, w0_v, ssem0)
    drain_scatter(rows1_v, w1_v, ssem1)
    plsc.subcore_barrier()

    def fchunk(b, carry):
        ci = b * 16 + sid

        @pl.when(ci < NZCHUNK)
        def _():
            pltpu.async_copy(accf_sh.at[pl.ds(ci * ZR, ZR)],
                             accf_hbm.at[cid, pl.ds(ci * ZR, ZR)], ssem0)
            pltpu.async_copy(accd_sh.at[pl.ds(ci * ZR, ZR)],
                             accd_hbm.at[cid, pl.ds(ci * ZR, ZR)], ssem1)
        return carry

    def fwait(b, carry):
        ci = b * 16 + sid

        @pl.when(ci < NZCHUNK)
        def _():
            pltpu.make_async_copy(
                accf_sh.at[pl.ds(ci * ZR, ZR)],
                accf_hbm.at[cid, pl.ds(ci * ZR, ZR)], ssem0).wait()
            pltpu.make_async_copy(
                accd_sh.at[pl.ds(ci * ZR, ZR)],
                accd_hbm.at[cid, pl.ds(ci * ZR, ZR)], ssem1).wait()
        return carry

    lax.fori_loop(0, (NZCHUNK + 15) // 16, fchunk, 0)
    lax.fori_loop(0, (NZCHUNK + 15) // 16, fwait, 0)


def _edge_pass(h, sd_flat, src, dst, mask, cvec, zf, zd):
    mesh = plsc.VectorSubcoreMesh(core_axis_name="c", subcore_axis_name="s")
    f = functools.partial(
        pl.kernel,
        mesh=mesh,
        out_type=(
            jax.ShapeDtypeStruct((2, N, D), jnp.float32),
            jax.ShapeDtypeStruct((2, N), jnp.float32),
        ),
        scratch_types=[
            pltpu.VMEM((2 * N,), jnp.float32),
            pltpu.VMEM((16,), jnp.float32),
            pltpu.VMEM((SBC, CH), jnp.int32),
            pltpu.VMEM((SBC, CH), jnp.int32),
            pltpu.VMEM((SBC, CH), jnp.float32),
            pltpu.VMEM((CH,), jnp.float32),
            pltpu.VMEM((CH,), jnp.float32),
            pltpu.VMEM((CH, D), jnp.float32),
            pltpu.VMEM((CH, D), jnp.float32),
            pltpu.VMEM_SHARED((N, D), jnp.float32),
            pltpu.VMEM_SHARED((N,), jnp.float32),
            pltpu.SemaphoreType.DMA,
            pltpu.SemaphoreType.DMA,
            pltpu.SemaphoreType.DMA,
            pltpu.SemaphoreType.DMA,
        ],
        compiler_params=pltpu.CompilerParams(use_tc_tiling_on_sc=False,
                                             needs_layout_passes=False),
    )(_edge_body)
    return f(h, sd_flat, src, dst, mask, cvec, zf, zd)


def _finish_body(accf_ref, accd_ref, out_ref):
    num = accf_ref[0] + accf_ref[1]
    den = accd_ref[0] + accd_ref[1]
    out_ref[...] = num / (den + 1e-15)


def _finish(accf, accd):
    return pl.pallas_call(
        _finish_body,
        grid=(N // BLK,),
        in_specs=[
            pl.BlockSpec((2, BLK, D), lambda i: (0, i, 0)),
            pl.BlockSpec((2, BLK, 1), lambda i: (0, i, 0)),
        ],
        out_specs=pl.BlockSpec((BLK, D), lambda i: (i, 0)),
        out_shape=jax.ShapeDtypeStruct((N, D), jnp.float32),
    )(accf, accd)


def kernel(x, edge_index, edge_mask, W, a_src, a_dst):
    a2 = jnp.stack([a_src, a_dst], axis=1)
    h, sd, cvec = _encode(x, W, a2)
    src = edge_index[0].reshape(E // CH, CH)
    dst = edge_index[1].reshape(E // CH, CH)
    mask2 = edge_mask.reshape(E // CH, CH)
    zf = jnp.zeros((ZR, D), jnp.float32)
    zd = jnp.zeros((ZR,), jnp.float32)
    accf, accd = _edge_pass(h, sd.reshape(2 * N), src, dst, mask2,
                            cvec.reshape(16), zf, zd)
    return _finish(accf, accd.reshape(2, N, 1))


# speedup vs baseline: 23.3743x; 1.0030x over previous
import functools

import jax
import jax.numpy as jnp
from jax import lax
from jax.experimental import pallas as pl
from jax.experimental.pallas import tpu as pltpu
from jax.experimental.pallas import tpu_sc as plsc

N = 10000
E = 320000
D = 128

DEN_W = 16
NW = 32
EPW = E // NW
CH = 80
NCHUNK = EPW // CH
ZR = 200
NZCHUNK = N // ZR
BLK = 1000


def _encode_body(x_ref, w_ref, a_ref, h_ref, sd_ref, c_ref, mx_ref):
    i = pl.program_id(0)
    h = jnp.dot(x_ref[...], w_ref[...], preferred_element_type=jnp.float32)
    h_ref[...] = h
    sd = jnp.dot(h, a_ref[...], preferred_element_type=jnp.float32)
    sd_ref[...] = sd
    ms = jnp.max(sd[:, 0])
    md = jnp.max(sd[:, 1])

    @pl.when(i == 0)
    def _():
        mx_ref[0] = ms
        mx_ref[1] = md

    @pl.when(i > 0)
    def _():
        mx_ref[0] = jnp.maximum(mx_ref[0], ms)
        mx_ref[1] = jnp.maximum(mx_ref[1], md)

    @pl.when(i == pl.num_programs(0) - 1)
    def _():
        c_ref[...] = jnp.full((1, 16), jnp.maximum(mx_ref[0] + mx_ref[1], 0.0),
                              jnp.float32)


def _encode(x, W, a2):
    return pl.pallas_call(
        _encode_body,
        grid=(N // BLK,),
        in_specs=[
            pl.BlockSpec((BLK, D), lambda i: (i, 0)),
            pl.BlockSpec((D, D), lambda i: (0, 0)),
            pl.BlockSpec((D, 2), lambda i: (0, 0)),
        ],
        out_specs=[
            pl.BlockSpec((BLK, D), lambda i: (i, 0)),
            pl.BlockSpec((BLK, 2), lambda i: (i, 0)),
            pl.BlockSpec((1, 16), lambda i: (0, 0)),
        ],
        out_shape=[
            jax.ShapeDtypeStruct((N, D), jnp.float32),
            jax.ShapeDtypeStruct((N, 2), jnp.float32),
            jax.ShapeDtypeStruct((1, 16), jnp.float32),
        ],
        scratch_shapes=[pltpu.SMEM((2,), jnp.float32)],
    )(x, W, a2)


SBC = 25
NBLK = NCHUNK // SBC


def _edge_body(h_hbm, sd_hbm, src_hbm, dst_hbm, mask_hbm, c_hbm, zf_hbm,
               zd_hbm, accf_hbm, accd_hbm, sd_v, c_v, srcb_v, dstb_v,
               maskb_v, w0_v, w1_v, rows0_v, rows1_v, accf_sh, accd_sh,
               gsem0, gsem1, ssem0, ssem1):
    cid = lax.axis_index("c")
    sid = lax.axis_index("s")
    wid = sid * 2 + cid

    pltpu.async_copy(sd_hbm, sd_v, gsem0)
    pltpu.async_copy(c_hbm, c_v, gsem1)

    def zchunk(b, carry):
        ci = b * 16 + sid

        @pl.when(ci < NZCHUNK)
        def _():
            pltpu.async_copy(zf_hbm, accf_sh.at[pl.ds(ci * ZR, ZR)], ssem0)
            pltpu.async_copy(zd_hbm, accd_sh.at[pl.ds(ci * ZR, ZR)], ssem1)
        return carry

    def zwait(b, carry):
        ci = b * 16 + sid

        @pl.when(ci < NZCHUNK)
        def _():
            pltpu.make_async_copy(
                zf_hbm, accf_sh.at[pl.ds(ci * ZR, ZR)], ssem0).wait()
            pltpu.make_async_copy(
                zd_hbm, accd_sh.at[pl.ds(ci * ZR, ZR)], ssem1).wait()
        return carry

    lax.fori_loop(0, (NZCHUNK + 15) // 16, zchunk, 0)
    lax.fori_loop(0, (NZCHUNK + 15) // 16, zwait, 0)
    pltpu.make_async_copy(sd_hbm, sd_v, gsem0).wait()
    pltpu.make_async_copy(c_hbm, c_v, gsem1).wait()
    plsc.subcore_barrier()

    cvec = c_v[...]
    crow0 = wid * NCHUNK

    def fire_gather(lci, rows_v, gsem):
        pltpu.async_copy(h_hbm.at[srcb_v.at[lci]], rows_v, gsem)

    def drain_scatter(rows_v, w_v, ssem):
        pltpu.make_async_copy(rows_v, accf_sh.at[dstb_v.at[0]], ssem).wait()
        pltpu.make_async_copy(w_v, accd_sh.at[dstb_v.at[0]], ssem).wait()

    def process(lci, rows_v, w_v, gsem, ssem):
        for g in range(CH // 16):
            s16 = srcb_v[lci, pl.ds(g * 16, 16)]
            d16 = dstb_v[lci, pl.ds(g * 16, 16)]
            mk = maskb_v[lci, pl.ds(g * 16, 16)]
            sv = plsc.load_gather(sd_v, [s16 * 2])
            dv = plsc.load_gather(sd_v, [d16 * 2 + 1])
            e = sv + dv
            e = jnp.where(e >= 0.0, e, 0.2 * e)
            w16 = jnp.exp(e - cvec) / (1.0 + jnp.exp(-mk))
            w_v[pl.ds(g * 16, 16)] = w16
        pltpu.make_async_copy(h_hbm.at[srcb_v.at[lci]], rows_v, gsem).wait()
        for g in range(CH // 16):
            w16 = w_v[pl.ds(g * 16, 16)]
            for j in range(16):
                i = g * 16 + j
                wb = w16[j]
                for k in range(D // 16):
                    rows_v[i, pl.ds(k * 16, 16)] = (
                        rows_v[i, pl.ds(k * 16, 16)] * wb)
        pltpu.async_copy(rows_v, accf_sh.at[dstb_v.at[lci]], ssem, add=True)
        pltpu.async_copy(w_v, accd_sh.at[dstb_v.at[lci]], ssem, add=True)

    def block(blk, carry):
        @pl.when(blk > 0)
        def _():
            drain_scatter(rows0_v, w0_v, ssem0)
            drain_scatter(rows1_v, w1_v, ssem1)

        brow = crow0 + blk * SBC
        pltpu.sync_copy(src_hbm.at[pl.ds(brow, SBC)], srcb_v)
        pltpu.sync_copy(dst_hbm.at[pl.ds(brow, SBC)], dstb_v)
        pltpu.sync_copy(mask_hbm.at[pl.ds(brow, SBC)], maskb_v)

        def pair(p, c2):
            lci0 = p * 2
            lci1 = p * 2 + 1
            notfirst = p > 0

            @pl.when(notfirst)
            def _():
                drain_scatter(rows0_v, w0_v, ssem0)
            fire_gather(lci0, rows0_v, gsem0)

            @pl.when(notfirst)
            def _():
                drain_scatter(rows1_v, w1_v, ssem1)
            fire_gather(lci1, rows1_v, gsem1)

            process(lci0, rows0_v, w0_v, gsem0, ssem0)
            process(lci1, rows1_v, w1_v, gsem1, ssem1)
            return c2

        lax.fori_loop(0, SBC // 2, pair, 0)

        drain_scatter(rows0_v, w0_v, ssem0)
        fire_gather(SBC - 1, rows0_v, gsem0)
        process(SBC - 1, rows0_v, w0_v, gsem0, ssem0)
        return carry

    lax.fori_loop(0, NBLK, block, 0)
    drain_scatter(rows0_v, w0_v, ssem0)
    drain_scatter(rows1_v, w1_v, ssem1)
    plsc.subcore_barrier()

    def fchunk(b, carry):
        ci = b * 16 + sid

        @pl.when(ci < NZCHUNK)
        def _():
            pltpu.async_copy(accf_sh.at[pl.ds(ci * ZR, ZR)],
                             accf_hbm.at[cid, pl.ds(ci * ZR, ZR)], ssem0)
            pltpu.async_copy(accd_sh.at[pl.ds(ci * ZR, ZR)],
                             accd_hbm.at[cid, pl.ds(ci * ZR, ZR)], ssem1)
        return carry

    def fwait(b, carry):
        ci = b * 16 + sid

        @pl.when(ci < NZCHUNK)
        def _():
            pltpu.make_async_copy(
                accf_sh.at[pl.ds(ci * ZR, ZR)],
                accf_hbm.at[cid, pl.ds(ci * ZR, ZR)], ssem0).wait()
            pltpu.make_async_copy(
                accd_sh.at[pl.ds(ci * ZR, ZR)],
                accd_hbm.at[cid, pl.ds(ci * ZR, ZR)], ssem1).wait()
        return carry

    lax.fori_loop(0, (NZCHUNK + 15) // 16, fchunk, 0)
    lax.fori_loop(0, (NZCHUNK + 15) // 16, fwait, 0)


def _edge_pass(h, sd_flat, src, dst, mask, cvec, zf, zd):
    mesh = plsc.VectorSubcoreMesh(core_axis_name="c", subcore_axis_name="s")
    f = functools.partial(
        pl.kernel,
        mesh=mesh,
        out_type=(
            jax.ShapeDtypeStruct((2, N, D), jnp.float32),
            jax.ShapeDtypeStruct((2, N), jnp.float32),
        ),
        scratch_types=[
            pltpu.VMEM((2 * N,), jnp.float32),
            pltpu.VMEM((16,), jnp.float32),
            pltpu.VMEM((SBC, CH), jnp.int32),
            pltpu.VMEM((SBC, CH), jnp.int32),
            pltpu.VMEM((SBC, CH), jnp.float32),
            pltpu.VMEM((CH,), jnp.float32),
            pltpu.VMEM((CH,), jnp.float32),
            pltpu.VMEM((CH, D), jnp.float32),
            pltpu.VMEM((CH, D), jnp.float32),
            pltpu.VMEM_SHARED((N, D), jnp.float32),
            pltpu.VMEM_SHARED((N,), jnp.float32),
            pltpu.SemaphoreType.DMA,
            pltpu.SemaphoreType.DMA,
            pltpu.SemaphoreType.DMA,
            pltpu.SemaphoreType.DMA,
        ],
        compiler_params=pltpu.CompilerParams(use_tc_tiling_on_sc=False,
                                             needs_layout_passes=False),
    )(_edge_body)
    return f(h, sd_flat, src, dst, mask, cvec, zf, zd)


def _finish_body(accf_ref, accd_ref, out_ref):
    num = accf_ref[0] + accf_ref[1]
    den = accd_ref[0] + accd_ref[1]
    out_ref[...] = num / (den + 1e-15)


def _finish(accf, accd):
    return pl.pallas_call(
        _finish_body,
        grid=(N // BLK,),
        in_specs=[
            pl.BlockSpec((2, BLK, D), lambda i: (0, i, 0)),
            pl.BlockSpec((2, BLK, 1), lambda i: (0, i, 0)),
        ],
        out_specs=pl.BlockSpec((BLK, D), lambda i: (i, 0)),
        out_shape=jax.ShapeDtypeStruct((N, D), jnp.float32),
    )(accf, accd)


def kernel(x, edge_index, edge_mask, W, a_src, a_dst):
    a2 = jnp.stack([a_src, a_dst], axis=1)
    h, sd, cvec = _encode(x, W, a2)
    src = edge_index[0].reshape(E // CH, CH)
    dst = edge_index[1].reshape(E // CH, CH)
    mask2 = edge_mask.reshape(E // CH, CH)
    zf = jnp.zeros((ZR, D), jnp.float32)
    zd = jnp.zeros((ZR,), jnp.float32)
    accf, accd = _edge_pass(h, sd.reshape(2 * N), src, dst, mask2,
                            cvec.reshape(16), zf, zd)
    return _finish(accf, accd.reshape(2, N, 1))
